# baseline (speedup 1.0000x reference)
import functools

import jax
import jax.numpy as jnp
from jax import lax
from jax.experimental import pallas as pl
from jax.experimental.pallas import tpu as pltpu
from jax.experimental.pallas import tpu_sc as plsc

N_FIELDS = 26
VOCAB = 100000
EMB = 16
OHE = 100
HID = 128
BATCH = 16384
EMB_FEAT = N_FIELDS * EMB
TOTAL_ROWS = BATCH * N_FIELDS

_NC = 2
_NS = 16
_NW = _NC * _NS
_UNIT = 1024
_UNITS_PER_FIELD = BATCH // _UNIT
_N_UNITS = N_FIELDS * _UNITS_PER_FIELD
_UNITS_PER_W = _N_UNITS // _NW


def _gather_body(table_hbm, idx_hbm, oidx_hbm, out_hbm, idx_v, oidx_v, rows_v, sem):
    wid = lax.axis_index("s") * _NC + lax.axis_index("c")
    for k in range(_UNITS_PER_W):
        u = wid * _UNITS_PER_W + k
        f = u // _UNITS_PER_FIELD
        off = u * _UNIT
        pltpu.sync_copy(idx_hbm.at[pl.ds(off, _UNIT)], idx_v)
        pltpu.sync_copy(oidx_hbm.at[pl.ds(off, _UNIT)], oidx_v)
        pltpu.async_copy(table_hbm.at[f].at[idx_v], rows_v, sem).wait()
        pltpu.async_copy(rows_v, out_hbm.at[oidx_v], sem).wait()


_gather = functools.partial(
    pl.kernel,
    mesh=plsc.VectorSubcoreMesh(core_axis_name="c", subcore_axis_name="s"),
    out_type=jax.ShapeDtypeStruct((TOTAL_ROWS, EMB), jnp.float32),
    scratch_types=[
        pltpu.VMEM((_UNIT,), jnp.int32),
        pltpu.VMEM((_UNIT,), jnp.int32),
        pltpu.VMEM((_UNIT, EMB), jnp.float32),
        pltpu.SemaphoreType.DMA,
    ],
    compiler_params=pltpu.CompilerParams(use_tc_tiling_on_sc=False),
)(_gather_body)


_BM = 2048


def _mm_body(e_ref, o_ref, w1_ref, w2_ref, b_ref, out_ref):
    acc = jnp.dot(e_ref[...], w1_ref[...], preferred_element_type=jnp.float32)
    acc = acc + jnp.dot(o_ref[...], w2_ref[...], preferred_element_type=jnp.float32)
    out_ref[...] = acc + b_ref[...]


_mm = pl.pallas_call(
    _mm_body,
    grid=(BATCH // _BM,),
    in_specs=[
        pl.BlockSpec((_BM, EMB_FEAT), lambda i: (i, 0)),
        pl.BlockSpec((_BM, OHE), lambda i: (i, 0)),
        pl.BlockSpec((EMB_FEAT, HID), lambda i: (0, 0)),
        pl.BlockSpec((OHE, HID), lambda i: (0, 0)),
        pl.BlockSpec((1, HID), lambda i: (0, 0)),
    ],
    out_specs=pl.BlockSpec((_BM, HID), lambda i: (i, 0)),
    out_shape=jax.ShapeDtypeStruct((BATCH, HID), jnp.float32),
)


@jax.jit
def kernel(embed_idx, ohes, tables, W, b):
    idx_fm = embed_idx.astype(jnp.int32).T.reshape(TOTAL_ROWS)
    oidx = (
        jnp.arange(BATCH, dtype=jnp.int32)[None, :] * N_FIELDS
        + jnp.arange(N_FIELDS, dtype=jnp.int32)[:, None]
    ).reshape(TOTAL_ROWS)
    e = _gather(tables, idx_fm, oidx)
    e = e.reshape(BATCH, EMB_FEAT)
    return _mm(e, ohes, W[:EMB_FEAT], W[EMB_FEAT:], b.reshape(1, HID))
